# baseline (speedup 1.0000x reference)
import functools

import jax
import jax.numpy as jnp
from jax import lax
from jax.experimental import pallas as pl
from jax.experimental.pallas import tpu as pltpu
from jax.experimental.pallas import tpu_sc as plsc

HIDDEN = 1024
INTER = 2048
NUM_EXPERTS = 8
TOP_K = 2
AUX_COEF = 0.001
T = 4096
ASSIGN = T * TOP_K

BM = 512
ROWS = ASSIGN + NUM_EXPERTS * BM
NB = ROWS // BM
NB1 = NB + 1
INT_BLK = 1024
NI = INTER // INT_BLK

RCH = 1024
SCH = 128

NW = 32
TPW = T // NW
DCHUNK = 32
CCHUNK = 16


def _router_kernel(x_ref, wgate_ref,
                   pos0_ref, pos1_ref, w0_ref, w1_ref, bexp_ref, aux_ref,
                   e0_ref, e1_ref, h_ref, s_ref):
    wgate = wgate_ref[...]
    psum = jnp.zeros((1, NUM_EXPERTS), jnp.float32)

    g8 = (lax.broadcasted_iota(jnp.int32, (RCH // SCH, RCH), 1) // SCH
          == lax.broadcasted_iota(jnp.int32, (RCH // SCH, RCH), 0)
          ).astype(jnp.float32)

    for c in range(T // RCH):
        rows = pl.ds(c * RCH, RCH)
        xb = x_ref[rows, :]
        logits = lax.dot_general(xb, wgate, (((1,), (1,)), ((), ())),
                                 preferred_element_type=jnp.float32)
        m = jnp.max(logits, axis=1, keepdims=True)
        ex = jnp.exp(logits - m)
        probs = ex / jnp.sum(ex, axis=1, keepdims=True)

        eidx = lax.broadcasted_iota(jnp.int32, probs.shape, 1)
        p0 = jnp.max(probs, axis=1, keepdims=True)
        e0 = jnp.min(jnp.where(probs == p0, eidx, NUM_EXPERTS),
                     axis=1, keepdims=True)
        oh0 = (eidx == e0).astype(jnp.float32)
        masked = jnp.where(eidx == e0, -jnp.inf, probs)
        p1 = jnp.max(masked, axis=1, keepdims=True)
        e1 = jnp.min(jnp.where(masked == p1, eidx, NUM_EXPERTS),
                     axis=1, keepdims=True)
        oh1 = (eidx == e1).astype(jnp.float32)

        denom = p0 + p1
        w0_ref[rows, :] = p0 / denom
        w1_ref[rows, :] = p1 / denom
        e0_ref[rows, :] = e0
        e1_ref[rows, :] = e1

        h = oh0 + oh1
        h_ref[rows, :] = h
        s_ref[pl.ds(c * (RCH // SCH), RCH // SCH), :] = lax.dot_general(
            g8, h, (((1,), (0,)), ((), ())),
            preferred_element_type=jnp.float32)
        psum = psum + jnp.sum(probs * h, axis=0, keepdims=True)

    s = s_ref[...]
    nsc = T // SCH
    slt32 = (lax.broadcasted_iota(jnp.int32, (nsc, nsc), 0)
             > lax.broadcasted_iota(jnp.int32, (nsc, nsc), 1)
             ).astype(jnp.float32)
    p32 = lax.dot_general(slt32, s, (((1,), (0,)), ((), ())),
                          preferred_element_type=jnp.float32)
    counts = jnp.sum(s, axis=0, keepdims=True)
    cnt_i = counts.astype(jnp.int32)
    pc = ((cnt_i + (BM - 1)) // BM) * BM
    sut = (lax.broadcasted_iota(jnp.int32, (NUM_EXPERTS, NUM_EXPERTS), 0)
           < lax.broadcasted_iota(jnp.int32, (NUM_EXPERTS, NUM_EXPERTS), 1)
           ).astype(jnp.float32)
    startsf = lax.dot_general(pc.astype(jnp.float32), sut,
                              (((1,), (0,)), ((), ())),
                              preferred_element_type=jnp.float32)
    starts = startsf.astype(jnp.int32)
    nact = jnp.sum(pc) // BM

    bs = lax.broadcasted_iota(jnp.int32, (NB1, NUM_EXPERTS), 0) * BM
    eix = lax.broadcasted_iota(jnp.int32, (NB1, NUM_EXPERTS), 1)
    hit = jnp.logical_and(bs >= starts, bs < starts + pc).astype(jnp.int32)
    elast = jnp.max(jnp.where(counts > 0, eix[:1, :], -1))
    rowi = lax.broadcasted_iota(jnp.int32, (NB1, 1), 0)
    bexp = jnp.sum(eix * hit, axis=1, keepdims=True)
    bexp = jnp.where(rowi < nact, bexp, elast)
    bexp = jnp.where(rowi == NB, nact, bexp)
    bexp_ref[...] = bexp

    slt = (lax.broadcasted_iota(jnp.int32, (SCH, SCH), 0)
           > lax.broadcasted_iota(jnp.int32, (SCH, SCH), 1)
           ).astype(jnp.float32)
    for c in range(nsc):
        rows = pl.ds(c * SCH, SCH)
        intra = lax.dot_general(slt, h_ref[rows, :], (((1,), (0,)), ((), ())),
                                preferred_element_type=jnp.float32)
        cums = intra + lax.slice(p32, (c, 0), (c + 1, NUM_EXPERTS)) + startsf
        eidx = lax.broadcasted_iota(jnp.int32, (SCH, NUM_EXPERTS), 1)
        oh0 = (eidx == e0_ref[rows, :]).astype(jnp.float32)
        oh1 = (eidx == e1_ref[rows, :]).astype(jnp.float32)
        pos0_ref[rows, :] = jnp.sum(cums * oh0, axis=1,
                                    keepdims=True).astype(jnp.int32)
        pos1_ref[rows, :] = jnp.sum(cums * oh1, axis=1,
                                    keepdims=True).astype(jnp.int32)

    p_expert = psum / jnp.float32(T)
    p_tok = counts / jnp.float32(ASSIGN)
    aux_ref[0, 0] = jnp.sum(p_expert * p_tok) * NUM_EXPERTS * AUX_COEF


def _router(x2d, W_gate, interpret=False):
    return pl.pallas_call(
        _router_kernel,
        in_specs=[
            pl.BlockSpec((T, HIDDEN), lambda: (0, 0)),
            pl.BlockSpec((NUM_EXPERTS, HIDDEN), lambda: (0, 0)),
        ],
        out_specs=[
            pl.BlockSpec((T, 1), lambda: (0, 0)),
            pl.BlockSpec((T, 1), lambda: (0, 0)),
            pl.BlockSpec((T, 1), lambda: (0, 0)),
            pl.BlockSpec((T, 1), lambda: (0, 0)),
            pl.BlockSpec((NB1, 1), lambda: (0, 0)),
            pl.BlockSpec(memory_space=pltpu.SMEM),
        ],
        out_shape=[
            jax.ShapeDtypeStruct((T, 1), jnp.int32),
            jax.ShapeDtypeStruct((T, 1), jnp.int32),
            jax.ShapeDtypeStruct((T, 1), jnp.float32),
            jax.ShapeDtypeStruct((T, 1), jnp.float32),
            jax.ShapeDtypeStruct((NB1, 1), jnp.int32),
            jax.ShapeDtypeStruct((1, 1), jnp.float32),
        ],
        scratch_shapes=[
            pltpu.VMEM((T, 1), jnp.int32),
            pltpu.VMEM((T, 1), jnp.int32),
            pltpu.VMEM((T, NUM_EXPERTS), jnp.float32),
            pltpu.VMEM((T // SCH, NUM_EXPERTS), jnp.float32),
        ],
        interpret=interpret,
    )(x2d, W_gate)


def _sc_mesh():
    return plsc.VectorSubcoreMesh(core_axis_name="c", subcore_axis_name="s",
                                  num_cores=2, num_subcores=16)


DSUB = TPW // DCHUNK


def _dispatch_body(x_hbm, pos0_hbm, pos1_hbm, xs_hbm,
                   idx0_v, idx1_v, rows_v, lsem0, lsem1,
                   ssem0a, ssem0b, ssem1a, ssem1b):
    wid = lax.axis_index("s") * 2 + lax.axis_index("c")
    base = wid * TPW
    for sub in range(DSUB):
        pltpu.sync_copy(pos0_hbm.at[pl.ds(base + sub * DCHUNK, DCHUNK)],
                        idx0_v.at[sub])
        pltpu.sync_copy(pos1_hbm.at[pl.ds(base + sub * DCHUNK, DCHUNK)],
                        idx1_v.at[sub])

    lsems = [lsem0, lsem1]
    ssems = [(ssem0a, ssem0b), (ssem1a, ssem1b)]
    ldesc = [None, None]
    sdesc = [None, None]

    def load(sub):
        buf = sub % 2
        ldesc[buf] = pltpu.async_copy(
            x_hbm.at[pl.ds(base + sub * DCHUNK, DCHUNK)],
            rows_v.at[buf], lsems[buf])

    load(0)
    for sub in range(DSUB):
        buf = sub % 2
        if sub + 1 < DSUB:
            if sdesc[1 - buf] is not None:
                sdesc[1 - buf][0].wait()
                sdesc[1 - buf][1].wait()
            load(sub + 1)
        ldesc[buf].wait()
        d0 = pltpu.async_copy(rows_v.at[buf], xs_hbm.at[idx0_v.at[sub]],
                              ssems[buf][0])
        d1 = pltpu.async_copy(rows_v.at[buf], xs_hbm.at[idx1_v.at[sub]],
                              ssems[buf][1])
        sdesc[buf] = (d0, d1)
    for pair in sdesc:
        if pair is not None:
            pair[0].wait()
            pair[1].wait()


@functools.lru_cache(maxsize=None)
def _make_dispatch():
    return pl.kernel(
        _dispatch_body,
        out_type=jax.ShapeDtypeStruct((ROWS, HIDDEN), jnp.float32),
        mesh=_sc_mesh(),
        scratch_types=[
            pltpu.VMEM((DSUB, DCHUNK), jnp.int32),
            pltpu.VMEM((DSUB, DCHUNK), jnp.int32),
            pltpu.VMEM((2, DCHUNK, HIDDEN), jnp.float32),
            pltpu.SemaphoreType.DMA,
            pltpu.SemaphoreType.DMA,
            pltpu.SemaphoreType.DMA,
            pltpu.SemaphoreType.DMA,
            pltpu.SemaphoreType.DMA,
            pltpu.SemaphoreType.DMA,
        ],
    )


def _ffn_kernel(bexp_ref, xs_ref, wg_ref, wu_ref, wd_ref, ys_ref):
    b = pl.program_id(0)
    i = pl.program_id(1)
    nblk = bexp_ref[NB]

    @pl.when(b < nblk)
    def _active():
        @pl.when(i == 0)
        def _init():
            ys_ref[...] = jnp.zeros_like(ys_ref)

        xb = xs_ref[...]
        g = lax.dot_general(xb, wg_ref[0], (((1,), (1,)), ((), ())),
                            preferred_element_type=jnp.float32)
        u = lax.dot_general(xb, wu_ref[0], (((1,), (1,)), ((), ())),
                            preferred_element_type=jnp.float32)
        h = (g * jax.nn.sigmoid(g)) * u
        ys_ref[...] += lax.dot_general(h, wd_ref[0], (((1,), (1,)), ((), ())),
                                       preferred_element_type=jnp.float32)


def _row_clamp(b, be):
    return jnp.minimum(b, be[NB] - 1)


def _i_clamp(b, i, be):
    nblk = be[NB]
    i_act = jnp.where(b % 2 == 1, NI - 1 - i, i)
    i_dead = jnp.where((nblk - 1) % 2 == 1, 0, NI - 1)
    return jnp.where(b < nblk, i_act, i_dead)


def _ffn(bexp, xs, Wg, Wu, Wd, interpret=False):
    grid_spec = pltpu.PrefetchScalarGridSpec(
        num_scalar_prefetch=1,
        grid=(NB, NI),
        in_specs=[
            pl.BlockSpec((BM, HIDDEN), lambda b, i, be: (_row_clamp(b, be), 0)),
            pl.BlockSpec((1, INT_BLK, HIDDEN),
                         lambda b, i, be: (be[b], _i_clamp(b, i, be), 0)),
            pl.BlockSpec((1, INT_BLK, HIDDEN),
                         lambda b, i, be: (be[b], _i_clamp(b, i, be), 0)),
            pl.BlockSpec((1, HIDDEN, INT_BLK),
                         lambda b, i, be: (be[b], 0, _i_clamp(b, i, be))),
        ],
        out_specs=pl.BlockSpec((BM, HIDDEN),
                               lambda b, i, be: (_row_clamp(b, be), 0)),
    )
    return pl.pallas_call(
        _ffn_kernel,
        grid_spec=grid_spec,
        out_shape=jax.ShapeDtypeStruct((ROWS, HIDDEN), jnp.float32),
        interpret=interpret,
    )(bexp, xs, Wg, Wu, Wd)


CSUB = TPW // CCHUNK


def _combine_body(ys_hbm, pos0_hbm, pos1_hbm, w0_hbm, w1_hbm, out_hbm,
                  idx0_v, idx1_v, w0_v, w1_v, a_v, b_v,
                  ga0, ga1, gb0, gb1, wbs0, wbs1):
    wid = lax.axis_index("s") * 2 + lax.axis_index("c")
    base = wid * TPW
    pltpu.sync_copy(pos0_hbm.at[pl.ds(base, TPW)], idx0_v)
    pltpu.sync_copy(pos1_hbm.at[pl.ds(base, TPW)], idx1_v)
    pltpu.sync_copy(w0_hbm.at[pl.ds(base, TPW)], w0_v)
    pltpu.sync_copy(w1_hbm.at[pl.ds(base, TPW)], w1_v)

    gsems = [(ga0, gb0), (ga1, gb1)]
    wsems = [wbs0, wbs1]
    gdesc = [None, None]
    wdesc = [None, None]

    def gather(sub):
        buf = sub % 2
        d0 = pltpu.async_copy(ys_hbm.at[idx0_v.at[pl.ds(sub * CCHUNK, CCHUNK)]],
                              a_v.at[buf], gsems[buf][0])
        d1 = pltpu.async_copy(ys_hbm.at[idx1_v.at[pl.ds(sub * CCHUNK, CCHUNK)]],
                              b_v.at[buf], gsems[buf][1])
        gdesc[buf] = (d0, d1)

    gather(0)
    for sub in range(CSUB):
        buf = sub % 2
        if sub + 1 < CSUB:
            if wdesc[1 - buf] is not None:
                wdesc[1 - buf].wait()
            gather(sub + 1)
        gdesc[buf][0].wait()
        gdesc[buf][1].wait()

        wv0 = w0_v[pl.ds(sub * CCHUNK, CCHUNK)]
        wv1 = w1_v[pl.ds(sub * CCHUNK, CCHUNK)]
        for i in range(CCHUNK):
            w0s = wv0[i]
            w1s = wv1[i]

            def col_body(j, _, buf=buf, i=i, w0s=w0s, w1s=w1s):
                for k in range(4):
                    sl = pl.ds(j * 64 + k * 16, 16)
                    a_v[buf, i, sl] = (a_v[buf, i, sl] * w0s
                                       + b_v[buf, i, sl] * w1s)
                return 0

            lax.fori_loop(0, HIDDEN // 64, col_body, 0)
        wdesc[buf] = pltpu.async_copy(
            a_v.at[buf], out_hbm.at[pl.ds(base + sub * CCHUNK, CCHUNK)],
            wsems[buf])
    for d in wdesc:
        if d is not None:
            d.wait()


@functools.lru_cache(maxsize=None)
def _make_combine():
    return pl.kernel(
        _combine_body,
        out_type=jax.ShapeDtypeStruct((T, HIDDEN), jnp.float32),
        mesh=_sc_mesh(),
        scratch_types=[
            pltpu.VMEM((TPW,), jnp.int32),
            pltpu.VMEM((TPW,), jnp.int32),
            pltpu.VMEM((TPW,), jnp.float32),
            pltpu.VMEM((TPW,), jnp.float32),
            pltpu.VMEM((2, CCHUNK, HIDDEN), jnp.float32),
            pltpu.VMEM((2, CCHUNK, HIDDEN), jnp.float32),
            pltpu.SemaphoreType.DMA,
            pltpu.SemaphoreType.DMA,
            pltpu.SemaphoreType.DMA,
            pltpu.SemaphoreType.DMA,
            pltpu.SemaphoreType.DMA,
            pltpu.SemaphoreType.DMA,
        ],
    )


@jax.jit
def _moe(x2d, W_gate, Wg, Wu, Wd):
    pos0, pos1, w0, w1, bexp, aux = _router(x2d, W_gate)
    p0 = pos0.reshape(T)
    p1 = pos1.reshape(T)
    xs = _make_dispatch()(x2d, p0, p1)
    ys = _ffn(bexp.reshape(NB1), xs, Wg, Wu, Wd)
    y2d = _make_combine()(ys, p0, p1, w0.reshape(T), w1.reshape(T))
    return y2d, aux[0, 0]


def kernel(x, W_gate, Wg, Wu, Wd):
    bsz, seq, hid = x.shape
    x2d = x.reshape(-1, hid)
    y, aux = _moe(x2d, W_gate, Wg, Wu, Wd)
    return y.reshape(bsz, seq, hid), aux


# speedup vs baseline: 1.0527x; 1.0527x over previous
import functools

import jax
import jax.numpy as jnp
from jax import lax
from jax.experimental import pallas as pl
from jax.experimental.pallas import tpu as pltpu
from jax.experimental.pallas import tpu_sc as plsc

HIDDEN = 1024
INTER = 2048
NUM_EXPERTS = 8
TOP_K = 2
AUX_COEF = 0.001
T = 4096
ASSIGN = T * TOP_K

BM = 512
ROWS = ASSIGN + NUM_EXPERTS * BM
NB = ROWS // BM
NB1 = NB + 1
INT_BLK = 1024
NI = INTER // INT_BLK

RCH = 1024
SCH = 128

NW = 32
TPW = T // NW
DCHUNK = 64
CCHUNK = 16


def _router_kernel(x_ref, wgate_ref,
                   pos0_ref, pos1_ref, w0_ref, w1_ref, bexp_ref, aux_ref,
                   e0_ref, e1_ref, h_ref, s_ref):
    wgate = wgate_ref[...]
    psum = jnp.zeros((1, NUM_EXPERTS), jnp.float32)

    g8 = (lax.broadcasted_iota(jnp.int32, (RCH // SCH, RCH), 1) // SCH
          == lax.broadcasted_iota(jnp.int32, (RCH // SCH, RCH), 0)
          ).astype(jnp.float32)

    for c in range(T // RCH):
        rows = pl.ds(c * RCH, RCH)
        xb = x_ref[rows, :]
        logits = lax.dot_general(xb, wgate, (((1,), (1,)), ((), ())),
                                 preferred_element_type=jnp.float32)
        m = jnp.max(logits, axis=1, keepdims=True)
        ex = jnp.exp(logits - m)
        probs = ex / jnp.sum(ex, axis=1, keepdims=True)

        eidx = lax.broadcasted_iota(jnp.int32, probs.shape, 1)
        p0 = jnp.max(probs, axis=1, keepdims=True)
        e0 = jnp.min(jnp.where(probs == p0, eidx, NUM_EXPERTS),
                     axis=1, keepdims=True)
        oh0 = (eidx == e0).astype(jnp.float32)
        masked = jnp.where(eidx == e0, -jnp.inf, probs)
        p1 = jnp.max(masked, axis=1, keepdims=True)
        e1 = jnp.min(jnp.where(masked == p1, eidx, NUM_EXPERTS),
                     axis=1, keepdims=True)
        oh1 = (eidx == e1).astype(jnp.float32)

        denom = p0 + p1
        w0_ref[rows, :] = p0 / denom
        w1_ref[rows, :] = p1 / denom
        e0_ref[rows, :] = e0
        e1_ref[rows, :] = e1

        h = oh0 + oh1
        h_ref[rows, :] = h
        s_ref[pl.ds(c * (RCH // SCH), RCH // SCH), :] = lax.dot_general(
            g8, h, (((1,), (0,)), ((), ())),
            preferred_element_type=jnp.float32)
        psum = psum + jnp.sum(probs * h, axis=0, keepdims=True)

    s = s_ref[...]
    nsc = T // SCH
    slt32 = (lax.broadcasted_iota(jnp.int32, (nsc, nsc), 0)
             > lax.broadcasted_iota(jnp.int32, (nsc, nsc), 1)
             ).astype(jnp.float32)
    p32 = lax.dot_general(slt32, s, (((1,), (0,)), ((), ())),
                          preferred_element_type=jnp.float32)
    counts = jnp.sum(s, axis=0, keepdims=True)
    cnt_i = counts.astype(jnp.int32)
    pc = ((cnt_i + (BM - 1)) // BM) * BM
    sut = (lax.broadcasted_iota(jnp.int32, (NUM_EXPERTS, NUM_EXPERTS), 0)
           < lax.broadcasted_iota(jnp.int32, (NUM_EXPERTS, NUM_EXPERTS), 1)
           ).astype(jnp.float32)
    startsf = lax.dot_general(pc.astype(jnp.float32), sut,
                              (((1,), (0,)), ((), ())),
                              preferred_element_type=jnp.float32)
    starts = startsf.astype(jnp.int32)
    nact = jnp.sum(pc) // BM

    bs = lax.broadcasted_iota(jnp.int32, (NB1, NUM_EXPERTS), 0) * BM
    eix = lax.broadcasted_iota(jnp.int32, (NB1, NUM_EXPERTS), 1)
    hit = jnp.logical_and(bs >= starts, bs < starts + pc).astype(jnp.int32)
    elast = jnp.max(jnp.where(counts > 0, eix[:1, :], -1))
    rowi = lax.broadcasted_iota(jnp.int32, (NB1, 1), 0)
    bexp = jnp.sum(eix * hit, axis=1, keepdims=True)
    bexp = jnp.where(rowi < nact, bexp, elast)
    bexp = jnp.where(rowi == NB, nact, bexp)
    bexp_ref[...] = bexp

    slt = (lax.broadcasted_iota(jnp.int32, (SCH, SCH), 0)
           > lax.broadcasted_iota(jnp.int32, (SCH, SCH), 1)
           ).astype(jnp.float32)
    for c in range(nsc):
        rows = pl.ds(c * SCH, SCH)
        intra = lax.dot_general(slt, h_ref[rows, :], (((1,), (0,)), ((), ())),
                                preferred_element_type=jnp.float32)
        cums = intra + lax.slice(p32, (c, 0), (c + 1, NUM_EXPERTS)) + startsf
        eidx = lax.broadcasted_iota(jnp.int32, (SCH, NUM_EXPERTS), 1)
        oh0 = (eidx == e0_ref[rows, :]).astype(jnp.float32)
        oh1 = (eidx == e1_ref[rows, :]).astype(jnp.float32)
        pos0_ref[rows, :] = jnp.sum(cums * oh0, axis=1,
                                    keepdims=True).astype(jnp.int32)
        pos1_ref[rows, :] = jnp.sum(cums * oh1, axis=1,
                                    keepdims=True).astype(jnp.int32)

    p_expert = psum / jnp.float32(T)
    p_tok = counts / jnp.float32(ASSIGN)
    aux_ref[0, 0] = jnp.sum(p_expert * p_tok) * NUM_EXPERTS * AUX_COEF


def _router(x2d, W_gate, interpret=False):
    return pl.pallas_call(
        _router_kernel,
        in_specs=[
            pl.BlockSpec((T, HIDDEN), lambda: (0, 0)),
            pl.BlockSpec((NUM_EXPERTS, HIDDEN), lambda: (0, 0)),
        ],
        out_specs=[
            pl.BlockSpec((T, 1), lambda: (0, 0)),
            pl.BlockSpec((T, 1), lambda: (0, 0)),
            pl.BlockSpec((T, 1), lambda: (0, 0)),
            pl.BlockSpec((T, 1), lambda: (0, 0)),
            pl.BlockSpec((NB1, 1), lambda: (0, 0)),
            pl.BlockSpec(memory_space=pltpu.SMEM),
        ],
        out_shape=[
            jax.ShapeDtypeStruct((T, 1), jnp.int32),
            jax.ShapeDtypeStruct((T, 1), jnp.int32),
            jax.ShapeDtypeStruct((T, 1), jnp.float32),
            jax.ShapeDtypeStruct((T, 1), jnp.float32),
            jax.ShapeDtypeStruct((NB1, 1), jnp.int32),
            jax.ShapeDtypeStruct((1, 1), jnp.float32),
        ],
        scratch_shapes=[
            pltpu.VMEM((T, 1), jnp.int32),
            pltpu.VMEM((T, 1), jnp.int32),
            pltpu.VMEM((T, NUM_EXPERTS), jnp.float32),
            pltpu.VMEM((T // SCH, NUM_EXPERTS), jnp.float32),
        ],
        interpret=interpret,
    )(x2d, W_gate)


def _sc_mesh():
    return plsc.VectorSubcoreMesh(core_axis_name="c", subcore_axis_name="s",
                                  num_cores=2, num_subcores=16)


def _dispatch_body(x_hbm, pos0_hbm, pos1_hbm, xs_hbm,
                   idx0_v, idx1_v, rows_v, sem0, sem1):
    wid = lax.axis_index("s") * 2 + lax.axis_index("c")
    for sub in range(TPW // DCHUNK):
        base = wid * TPW + sub * DCHUNK
        pltpu.sync_copy(pos0_hbm.at[pl.ds(base, DCHUNK)], idx0_v)
        pltpu.sync_copy(pos1_hbm.at[pl.ds(base, DCHUNK)], idx1_v)
        pltpu.sync_copy(x_hbm.at[pl.ds(base, DCHUNK)], rows_v)
        d0 = pltpu.async_copy(rows_v, xs_hbm.at[idx0_v], sem0)
        d1 = pltpu.async_copy(rows_v, xs_hbm.at[idx1_v], sem1)
        d0.wait()
        d1.wait()


@functools.lru_cache(maxsize=None)
def _make_dispatch():
    return pl.kernel(
        _dispatch_body,
        out_type=jax.ShapeDtypeStruct((ROWS, HIDDEN), jnp.float32),
        mesh=_sc_mesh(),
        scratch_types=[
            pltpu.VMEM((DCHUNK,), jnp.int32),
            pltpu.VMEM((DCHUNK,), jnp.int32),
            pltpu.VMEM((DCHUNK, HIDDEN), jnp.float32),
            pltpu.SemaphoreType.DMA,
            pltpu.SemaphoreType.DMA,
        ],
    )


def _ffn_kernel(bexp_ref, xs_ref, wg_ref, wu_ref, wd_ref, ys_ref, acc_ref):
    b = pl.program_id(0)
    i = pl.program_id(1)
    nblk = bexp_ref[NB]

    @pl.when(b < nblk)
    def _active():
        xb = xs_ref[...]
        g = lax.dot_general(xb, wg_ref[0], (((1,), (1,)), ((), ())),
                            preferred_element_type=jnp.float32)
        u = lax.dot_general(xb, wu_ref[0], (((1,), (1,)), ((), ())),
                            preferred_element_type=jnp.float32)
        h = (g * jax.nn.sigmoid(g)) * u
        part = lax.dot_general(h, wd_ref[0], (((1,), (1,)), ((), ())),
                               preferred_element_type=jnp.float32)

        @pl.when(i == 0)
        def _first():
            acc_ref[...] = part

        @pl.when(i > 0)
        def _rest():
            acc_ref[...] += part

        @pl.when(i == NI - 1)
        def _emit():
            ys_ref[...] = acc_ref[...]


def _row_clamp(b, be):
    return jnp.minimum(b, be[NB] - 1)


def _i_clamp(b, i, be):
    nblk = be[NB]
    i_act = jnp.where(b % 2 == 1, NI - 1 - i, i)
    i_dead = jnp.where((nblk - 1) % 2 == 1, 0, NI - 1)
    return jnp.where(b < nblk, i_act, i_dead)


def _ffn(bexp, xs, Wg, Wu, Wd, interpret=False):
    grid_spec = pltpu.PrefetchScalarGridSpec(
        num_scalar_prefetch=1,
        grid=(NB, NI),
        in_specs=[
            pl.BlockSpec((BM, HIDDEN), lambda b, i, be: (_row_clamp(b, be), 0)),
            pl.BlockSpec((1, INT_BLK, HIDDEN),
                         lambda b, i, be: (be[b], _i_clamp(b, i, be), 0)),
            pl.BlockSpec((1, INT_BLK, HIDDEN),
                         lambda b, i, be: (be[b], _i_clamp(b, i, be), 0)),
            pl.BlockSpec((1, HIDDEN, INT_BLK),
                         lambda b, i, be: (be[b], 0, _i_clamp(b, i, be))),
        ],
        out_specs=pl.BlockSpec((BM, HIDDEN),
                               lambda b, i, be: (_row_clamp(b, be), 0)),
        scratch_shapes=[pltpu.VMEM((BM, HIDDEN), jnp.float32)],
    )
    return pl.pallas_call(
        _ffn_kernel,
        grid_spec=grid_spec,
        out_shape=jax.ShapeDtypeStruct((ROWS, HIDDEN), jnp.float32),
        interpret=interpret,
    )(bexp, xs, Wg, Wu, Wd)


CSUB = TPW // CCHUNK


def _combine_body(ys_hbm, pos0_hbm, pos1_hbm, w0_hbm, w1_hbm, out_hbm,
                  idx0_v, idx1_v, w0_v, w1_v, a_v, b_v,
                  ga0, ga1, gb0, gb1, wbs0, wbs1):
    wid = lax.axis_index("s") * 2 + lax.axis_index("c")
    base = wid * TPW
    pltpu.sync_copy(pos0_hbm.at[pl.ds(base, TPW)], idx0_v)
    pltpu.sync_copy(pos1_hbm.at[pl.ds(base, TPW)], idx1_v)
    pltpu.sync_copy(w0_hbm.at[pl.ds(base, TPW)], w0_v)
    pltpu.sync_copy(w1_hbm.at[pl.ds(base, TPW)], w1_v)

    gsems = [(ga0, gb0), (ga1, gb1)]
    wsems = [wbs0, wbs1]
    gdesc = [None, None]
    wdesc = [None, None]

    def gather(sub):
        buf = sub % 2
        d0 = pltpu.async_copy(ys_hbm.at[idx0_v.at[pl.ds(sub * CCHUNK, CCHUNK)]],
                              a_v.at[buf], gsems[buf][0])
        d1 = pltpu.async_copy(ys_hbm.at[idx1_v.at[pl.ds(sub * CCHUNK, CCHUNK)]],
                              b_v.at[buf], gsems[buf][1])
        gdesc[buf] = (d0, d1)

    gather(0)
    for sub in range(CSUB):
        buf = sub % 2
        if sub + 1 < CSUB:
            if wdesc[1 - buf] is not None:
                wdesc[1 - buf].wait()
            gather(sub + 1)
        gdesc[buf][0].wait()
        gdesc[buf][1].wait()

        wv0 = w0_v[pl.ds(sub * CCHUNK, CCHUNK)]
        wv1 = w1_v[pl.ds(sub * CCHUNK, CCHUNK)]
        for i in range(CCHUNK):
            w0s = wv0[i]
            w1s = wv1[i]

            def col_body(j, _, buf=buf, i=i, w0s=w0s, w1s=w1s):
                for k in range(8):
                    sl = pl.ds(j * 128 + k * 16, 16)
                    a_v[buf, i, sl] = (a_v[buf, i, sl] * w0s
                                       + b_v[buf, i, sl] * w1s)
                return 0

            lax.fori_loop(0, HIDDEN // 128, col_body, 0)
        wdesc[buf] = pltpu.async_copy(
            a_v.at[buf], out_hbm.at[pl.ds(base + sub * CCHUNK, CCHUNK)],
            wsems[buf])
    for d in wdesc:
        if d is not None:
            d.wait()


@functools.lru_cache(maxsize=None)
def _make_combine():
    return pl.kernel(
        _combine_body,
        out_type=jax.ShapeDtypeStruct((T, HIDDEN), jnp.float32),
        mesh=_sc_mesh(),
        scratch_types=[
            pltpu.VMEM((TPW,), jnp.int32),
            pltpu.VMEM((TPW,), jnp.int32),
            pltpu.VMEM((TPW,), jnp.float32),
            pltpu.VMEM((TPW,), jnp.float32),
            pltpu.VMEM((2, CCHUNK, HIDDEN), jnp.float32),
            pltpu.VMEM((2, CCHUNK, HIDDEN), jnp.float32),
            pltpu.SemaphoreType.DMA,
            pltpu.SemaphoreType.DMA,
            pltpu.SemaphoreType.DMA,
            pltpu.SemaphoreType.DMA,
            pltpu.SemaphoreType.DMA,
            pltpu.SemaphoreType.DMA,
        ],
    )


@jax.jit
def _moe(x2d, W_gate, Wg, Wu, Wd):
    pos0, pos1, w0, w1, bexp, aux = _router(x2d, W_gate)
    p0 = pos0.reshape(T)
    p1 = pos1.reshape(T)
    xs = _make_dispatch()(x2d, p0, p1)
    ys = _ffn(bexp.reshape(NB1), xs, Wg, Wu, Wd)
    y2d = _make_combine()(ys, p0, p1, w0.reshape(T), w1.reshape(T))
    return y2d, aux[0, 0]


def kernel(x, W_gate, Wg, Wu, Wd):
    bsz, seq, hid = x.shape
    x2d = x.reshape(-1, hid)
    y, aux = _moe(x2d, W_gate, Wg, Wu, Wd)
    return y.reshape(bsz, seq, hid), aux
